# Initial kernel scaffold; baseline (speedup 1.0000x reference)
import jax
import jax.numpy as jnp
from jax.experimental import pallas as pl


def kernel(dense, sparse, sparse_table, dense_W, dense_b, bias):
    raise NotImplementedError("write your pallas kernel here")


# speedup vs baseline: 1.0502x; 1.0502x over previous
import functools

import jax
import jax.numpy as jnp
from jax import lax
from jax.experimental import pallas as pl
from jax.experimental.pallas import tpu as pltpu
from jax.experimental.pallas import tpu_sc as plsc

B = 16384
F = 26
ND = 13
NC = 2
NS = 16
NW = NC * NS
BPW = B // NW
IDXW = BPW * F
CHUNK = 128
NCHUNK = IDXW // CHUNK
FIRE = 8
NGROUP = BPW // 16


def _body(table_hbm, idx_hbm, dense_hbm, wv_hbm, out_hbm,
          idx_v, vals_v, dense_v, w_v, out_v, sem):
    wid = lax.axis_index("s") * NC + lax.axis_index("c")
    base = wid * BPW

    pltpu.sync_copy(idx_hbm.at[wid], idx_v)
    pltpu.sync_copy(dense_hbm.at[pl.ds(base * ND, BPW * ND)], dense_v)
    pltpu.sync_copy(wv_hbm, w_v)

    @pl.loop(0, NCHUNK, step=FIRE)
    def _gather(j0):
        copies = []
        for t in range(FIRE):
            j = j0 + t
            copies.append(
                pltpu.async_copy(
                    table_hbm.at[idx_v.at[j]],
                    vals_v.at[pl.ds(j * CHUNK, CHUNK)],
                    sem,
                )
            )
        for c in copies:
            c.wait()

    lane = jnp.arange(16, dtype=jnp.int32)
    w16 = w_v[...]
    wsplat = [
        jnp.take_along_axis(w16, jnp.full((16,), d, jnp.int32), axis=0)
        for d in range(ND + 1)
    ]

    @pl.loop(0, NGROUP)
    def _compute(g):
        acc = wsplat[ND]
        vbase = g * (16 * F) + lane * F
        for f in range(F):
            acc = acc + plsc.load_gather(vals_v, [vbase + f])
        dbase = g * (16 * ND) + lane * ND
        for d in range(ND):
            acc = acc + wsplat[d] * plsc.load_gather(dense_v, [dbase + d])
        out_v[pl.ds(g * 16, 16)] = acc

    pltpu.sync_copy(out_v, out_hbm.at[pl.ds(base, BPW)])


@jax.jit
def _lr_sc(table_flat, idx3, dense_flat, wv):
    mesh = plsc.VectorSubcoreMesh(core_axis_name="c", subcore_axis_name="s")
    return pl.kernel(
        _body,
        out_type=jax.ShapeDtypeStruct((B,), jnp.float32),
        mesh=mesh,
        compiler_params=pltpu.CompilerParams(needs_layout_passes=False),
        scratch_types=[
            pltpu.VMEM((NCHUNK, CHUNK), jnp.int32),
            pltpu.VMEM((IDXW,), jnp.float32),
            pltpu.VMEM((BPW * ND,), jnp.float32),
            pltpu.VMEM((16,), jnp.float32),
            pltpu.VMEM((BPW,), jnp.float32),
            pltpu.SemaphoreType.DMA,
        ],
    )(table_flat, idx3, dense_flat, wv)


def kernel(dense, sparse, sparse_table, dense_W, dense_b, bias):
    idx3 = sparse.astype(jnp.int32).reshape(NW, NCHUNK, CHUNK)
    table_flat = sparse_table.reshape(-1)
    dense_flat = dense.reshape(-1)
    wv = jnp.concatenate(
        [dense_W.reshape(-1),
         (dense_b + bias).reshape(-1),
         jnp.zeros(2, jnp.float32)]
    )
    return _lr_sc(table_flat, idx3, dense_flat, wv)


# speedup vs baseline: 1.1240x; 1.0703x over previous
import functools

import jax
import jax.numpy as jnp
from jax import lax
from jax.experimental import pallas as pl
from jax.experimental.pallas import tpu as pltpu
from jax.experimental.pallas import tpu_sc as plsc

B = 16384
F = 26
ND = 13
NC = 2
NS = 16
NW = NC * NS
BPW = B // NW
IDXW = BPW * F
CHUNK = 128
NCHUNK = IDXW // CHUNK
NSTAGE = 8
SCHUNK = NCHUNK // NSTAGE
SROWS = BPW // NSTAGE
SGROUP = SROWS // 16


def _body(table_hbm, idx_hbm, dense_hbm, wv_hbm, out_hbm,
          idx_v, vals_v, dense_v, w_v, out_v, sem0, sem1):
    wid = lax.axis_index("s") * NC + lax.axis_index("c")
    base = wid * BPW

    pltpu.sync_copy(idx_hbm.at[wid], idx_v)
    pltpu.sync_copy(dense_hbm.at[pl.ds(base * ND, BPW * ND)], dense_v)
    pltpu.sync_copy(wv_hbm, w_v)

    lane = jnp.arange(16, dtype=jnp.int32)
    w16 = w_v[...]
    wsplat = [
        jnp.take_along_axis(w16, jnp.full((16,), d, jnp.int32), axis=0)
        for d in range(ND + 1)
    ]
    sems = (sem0, sem1)

    def fire(s):
        sem = sems[s % 2]
        return [
            pltpu.async_copy(
                table_hbm.at[idx_v.at[s * SCHUNK + t]],
                vals_v.at[pl.ds((s * SCHUNK + t) * CHUNK, CHUNK)],
                sem,
            )
            for t in range(SCHUNK)
        ]

    def compute(s):
        @pl.loop(0, SGROUP)
        def _compute(gg):
            g = s * SGROUP + gg
            acc = wsplat[ND]
            vbase = g * (16 * F) + lane * F
            for f in range(F):
                acc = acc + plsc.load_gather(vals_v, [vbase + f])
            dbase = g * (16 * ND) + lane * ND
            for d in range(ND):
                acc = acc + wsplat[d] * plsc.load_gather(dense_v, [dbase + d])
            out_v[pl.ds(g * 16, 16)] = acc

    pending = fire(0)
    for s in range(NSTAGE):
        nxt = fire(s + 1) if s + 1 < NSTAGE else []
        for c in pending:
            c.wait()
        compute(s)
        pending = nxt

    pltpu.sync_copy(out_v, out_hbm.at[pl.ds(base, BPW)])


@jax.jit
def _lr_sc(table_flat, idx3, dense_flat, wv):
    mesh = plsc.VectorSubcoreMesh(core_axis_name="c", subcore_axis_name="s")
    return pl.kernel(
        _body,
        out_type=jax.ShapeDtypeStruct((B,), jnp.float32),
        mesh=mesh,
        compiler_params=pltpu.CompilerParams(needs_layout_passes=False),
        scratch_types=[
            pltpu.VMEM((NCHUNK, CHUNK), jnp.int32),
            pltpu.VMEM((IDXW,), jnp.float32),
            pltpu.VMEM((BPW * ND,), jnp.float32),
            pltpu.VMEM((16,), jnp.float32),
            pltpu.VMEM((BPW,), jnp.float32),
            pltpu.SemaphoreType.DMA,
            pltpu.SemaphoreType.DMA,
        ],
    )(table_flat, idx3, dense_flat, wv)


def kernel(dense, sparse, sparse_table, dense_W, dense_b, bias):
    idx3 = sparse.astype(jnp.int32).reshape(NW, NCHUNK, CHUNK)
    table_flat = sparse_table.reshape(-1)
    dense_flat = dense.reshape(-1)
    wv = jnp.concatenate(
        [dense_W.reshape(-1),
         (dense_b + bias).reshape(-1),
         jnp.zeros(2, jnp.float32)]
    )
    return _lr_sc(table_flat, idx3, dense_flat, wv)
